# baseline (speedup 1.0000x reference)
import functools

import jax
import jax.numpy as jnp
import numpy as np
from jax.experimental import pallas as pl
from jax.experimental.pallas import tpu as pltpu

H = 128
W = 128
NG = 1024
RB = 16
CB = 64
NRB = H // RB
NCHUNK = 2
NC = NG // NCHUNK
LOG2E = float(np.log2(np.e))


def _prologue_kernel(dataT_ref, op_ref, drv_ref):
    p = dataT_ref[0]
    x = jnp.tanh(p[0:1])
    y = jnp.tanh(p[1:2])
    xs = 0.5 * (x + 1.0) * W
    ys = 0.5 * (y + 1.0) * H
    sx = jnp.abs(p[2:3]) + 0.3
    sy = jnp.abs(p[3:4]) + 0.3
    theta = jax.nn.sigmoid(p[4:5]) * (2.0 * np.pi)
    cos = jnp.cos(theta)
    sin = jnp.sin(theta)
    sx2 = sx * sx
    sy2 = sy * sy
    sig_a = cos * cos * sx2 + sin * sin * sy2
    sig_b = cos * sin * (sx2 - sy2)
    sig_c = sin * sin * sx2 + cos * cos * sy2
    det = sig_a * sig_c - sig_b * sig_b
    inv_det = 1.0 / det
    aa = (-0.5 * LOG2E) * sig_c * inv_det
    gg = (-0.5 * LOG2E) * sig_a * inv_det
    bb = LOG2E * sig_b * inv_det
    colop = p[5:8] * op_ref[0:1]
    drv_ref[0] = jnp.concatenate([xs, ys, aa, gg, bb, colop], axis=0)


def _raster_kernel(feat_ref, drv_ref, out_ref):
    c = pl.program_id(2)
    jr = pl.program_id(1)

    d = drv_ref[0]
    cy = (jr * RB).astype(jnp.float32) + (RB // 2)
    yt = d[1:2] - cy
    aa = d[2:3]
    gg = d[3:4]
    bb = d[4:5]
    colop8 = jnp.concatenate(
        [d[5:8], jnp.zeros((5, NC), jnp.float32)],
        axis=0).astype(jnp.bfloat16).T

    ggyt = gg * yt
    bbyt = bb * yt
    kv_base = -2.0 * ggyt
    k_y = ggyt * yt

    @pl.when(c == 0)
    def _():
        out_ref[...] = jnp.zeros_like(out_ref)

    for half in range(2):
        cx = float(half * CB + CB // 2)
        xt = d[0:1] - cx
        aaxt = aa * xt
        k0 = aaxt * xt + k_y + (bb * xt) * yt
        ku = -2.0 * aaxt - bbyt
        kv = kv_base - bb * xt
        kmat = jnp.concatenate(
            [k0, ku, kv, aa, gg, bb, jnp.zeros((2, NC), jnp.float32)],
            axis=0)

        power2 = jax.lax.dot_general(
            feat_ref[...], kmat, (((1,), (0,)), ((), ())),
            preferred_element_type=jnp.float32)
        alpha = jnp.exp2(power2).astype(jnp.bfloat16)

        res = jax.lax.dot_general(
            alpha, colop8, (((1,), (0,)), ((), ())),
            preferred_element_type=jnp.float32)
        contrib = res.T.reshape(8, RB, CB)
        out_ref[0, :, :, half * CB:(half + 1) * CB] += contrib


@functools.partial(jax.jit, static_argnames=())
def kernel(data, opacity, background):
    bsz = data.shape[0]
    dataT = data.transpose(0, 2, 1)
    opT = opacity.reshape(1, NG)

    vv, uu = jnp.meshgrid(
        jnp.arange(RB, dtype=jnp.float32) - (RB // 2 - 0.5),
        jnp.arange(CB, dtype=jnp.float32) - (CB // 2 - 0.5),
        indexing="ij")
    u = uu.reshape(-1)
    v = vv.reshape(-1)
    feat = jnp.stack(
        [jnp.ones_like(u), u, v, u * u, v * v, u * v,
         jnp.zeros_like(u), jnp.zeros_like(u)], axis=1)

    derived = pl.pallas_call(
        _prologue_kernel,
        grid=(bsz,),
        in_specs=[
            pl.BlockSpec((1, 8, NG), lambda b: (b, 0, 0)),
            pl.BlockSpec((1, NG), lambda b: (0, 0)),
        ],
        out_specs=pl.BlockSpec((1, 8, NG), lambda b: (b, 0, 0)),
        out_shape=jax.ShapeDtypeStruct((bsz, 8, NG), jnp.float32),
    )(dataT, opT)

    out_pal = pl.pallas_call(
        _raster_kernel,
        grid=(bsz, NRB, NCHUNK),
        in_specs=[
            pl.BlockSpec((RB * CB, 8), lambda b, jr, c: (0, 0)),
            pl.BlockSpec((1, 8, NC), lambda b, jr, c: (b, 0, c)),
        ],
        out_specs=pl.BlockSpec(
            (1, 8, RB, W), lambda b, jr, c: (b, 0, jr, 0)),
        out_shape=jax.ShapeDtypeStruct((bsz, 8, H, W), jnp.float32),
        compiler_params=pltpu.CompilerParams(
            dimension_semantics=("parallel", "parallel", "arbitrary")),
    )(feat, derived)

    return out_pal[:, :3] + background[None, :, None, None]
